# baseline (speedup 1.0000x reference)
import functools

import jax
import jax.numpy as jnp
from jax import lax
from jax.experimental import pallas as pl
from jax.experimental.pallas import tpu as pltpu
from jax.experimental.pallas import tpu_sc as plsc

_C = 768
_K = 200
_SEL = 128
_S = 576
_SP = 640


def _compute_body(xt_ref, w_ref, b_ref, key_ref,
                  preds1_ref, preds0_ref, gidx_ref):
    n = pl.program_id(0)
    x_n = xt_ref[0]
    w = w_ref[...]
    logits = lax.dot_general(
        x_n, w, (((1,), (1,)), ((), ())),
        preferred_element_type=jnp.float32)
    logits = logits + b_ref[...]

    key_col = key_ref[0]

    keyp = jnp.concatenate(
        [key_col, jnp.full((_SP - _S, 1), -1e30, jnp.float32)], axis=0)

    ii = lax.broadcasted_iota(jnp.int32, (_SP, _SP), 0)
    jj = lax.broadcasted_iota(jnp.int32, (_SP, _SP), 1)
    eye = (ii == jj).astype(jnp.float32)
    key_row = lax.dot_general(
        keyp, eye, (((0,), (0,)), ((), ())),
        precision=lax.Precision.HIGHEST,
        preferred_element_type=jnp.float32)

    gt = key_row > keyp
    tie = (key_row == keyp) & (jj < ii)
    amat = (gt | tie).astype(jnp.float32)
    ones_col = jnp.ones((_SP, 1), jnp.float32)
    rank_f = lax.dot_general(
        amat, ones_col, (((1,), (0,)), ((), ())),
        preferred_element_type=jnp.float32)
    rank_col = rank_f.astype(jnp.int32)

    onehot = (rank_col == jj).astype(jnp.float32)

    iota_col = ii[:, :1].astype(jnp.float32)
    perm_col = lax.dot_general(
        onehot[:, :_SEL], iota_col, (((0,), (0,)), ((), ())),
        precision=lax.Precision.HIGHEST,
        preferred_element_type=jnp.float32)
    gidx_ref[...] = (perm_col.astype(jnp.int32)
                     + n * _S).reshape(1, _SEL, 1)

    sorted_logits = lax.dot_general(
        onehot[:_S, :_S], logits, (((0,), (0,)), ((), ())),
        preferred_element_type=jnp.float32)
    preds1_ref[...] = sorted_logits[:_SEL].reshape(1, _SEL, _K)
    preds0_ref[...] = sorted_logits[_SEL:].reshape(1, _S - _SEL, _K)


def _compute(xt, w, b2d, keys):
    n = xt.shape[0]
    return pl.pallas_call(
        _compute_body,
        grid=(n,),
        in_specs=[
            pl.BlockSpec((1, _S, _C), lambda i: (i, 0, 0)),
            pl.BlockSpec((_K, _C), lambda i: (0, 0)),
            pl.BlockSpec((1, _K), lambda i: (0, 0)),
            pl.BlockSpec((1, _S, 1), lambda i: (i, 0, 0)),
        ],
        out_specs=[
            pl.BlockSpec((1, _SEL, _K), lambda i: (i, 0, 0)),
            pl.BlockSpec((1, _S - _SEL, _K), lambda i: (i, 0, 0)),
            pl.BlockSpec((1, _SEL, 1), lambda i: (i, 0, 0)),
        ],
        out_shape=[
            jax.ShapeDtypeStruct((n, _SEL, _K), jnp.float32),
            jax.ShapeDtypeStruct((n, _S - _SEL, _K), jnp.float32),
            jax.ShapeDtypeStruct((n, _SEL, 1), jnp.int32),
        ],
    )(xt, w, b2d, keys)


_NW = 32
_BSEL = (16 * _SEL) // _NW


def _sc_gather(xt_flat, sel_idx):
    mesh = plsc.VectorSubcoreMesh(core_axis_name="c", subcore_axis_name="s")
    nsel = sel_idx.shape[0]

    @functools.partial(
        pl.kernel,
        mesh=mesh,
        out_type=jax.ShapeDtypeStruct((nsel, _C), jnp.float32),
        scratch_types=[
            pltpu.VMEM((_BSEL,), jnp.int32),
            pltpu.VMEM((_BSEL, _C), jnp.float32),
            pltpu.SemaphoreType.DMA,
        ],
    )
    def run(xt_hbm, sidx_hbm, sel_out, sidx_v, srows_v, sem_s):
        wid = lax.axis_index("s") * 2 + lax.axis_index("c")
        sbase = wid * _BSEL
        pltpu.sync_copy(sidx_hbm.at[pl.ds(sbase, _BSEL)], sidx_v)
        pltpu.async_copy(xt_hbm.at[sidx_v], srows_v, sem_s).wait()
        pltpu.sync_copy(srows_v, sel_out.at[pl.ds(sbase, _BSEL)])

    return run(xt_flat, sel_idx)


def kernel(x, W, b):
    n, c, h, wd = x.shape
    s = h * wd
    xt = jnp.transpose(x.reshape(n, c, s), (0, 2, 1))
    key_logits = xt @ W.T + b
    keys = jnp.max(jax.nn.softmax(key_logits, axis=-1), axis=-1)
    preds1, preds0, gidx = _compute(xt, W, b.reshape(1, _K),
                                    keys.reshape(n, s, 1))
    sel_idx = gidx.reshape(-1)
    sel_rows = _sc_gather(xt.reshape(n * s, c), sel_idx)
    selections = sel_rows.reshape(n, _SEL, c)
    return (selections, preds1, preds0)


# speedup vs baseline: 1.0503x; 1.0503x over previous
import functools

import jax
import jax.numpy as jnp
from jax import lax
from jax.experimental import pallas as pl
from jax.experimental.pallas import tpu as pltpu
from jax.experimental.pallas import tpu_sc as plsc

_C = 768
_K = 200
_SEL = 128
_S = 576
_SP = 640


def _compute_body(xt_ref, w_ref, b_ref, key_ref,
                  preds1_ref, preds0_ref, gidx_ref):
    n = pl.program_id(0)
    x_n = xt_ref[0]
    w = w_ref[...]
    logits = lax.dot_general(
        x_n, w, (((1,), (1,)), ((), ())),
        preferred_element_type=jnp.float32)
    logits = logits + b_ref[...]

    key_col = key_ref[0]

    keyp = jnp.concatenate(
        [key_col, jnp.full((_SP - _S, 1), -1e30, jnp.float32)], axis=0)

    ii = lax.broadcasted_iota(jnp.int32, (_SP, _SP), 0)
    jj = lax.broadcasted_iota(jnp.int32, (_SP, _SP), 1)
    eye = (ii == jj).astype(jnp.float32)
    key_row = lax.dot_general(
        keyp, eye, (((0,), (0,)), ((), ())),
        precision=lax.Precision.HIGHEST,
        preferred_element_type=jnp.float32)

    gt = key_row > keyp
    tie = (key_row == keyp) & (jj < ii)
    rank_col = jnp.sum((gt | tie).astype(jnp.float32), axis=1,
                       keepdims=True).astype(jnp.int32)

    onehot = (rank_col == jj).astype(jnp.float32)

    iota_i = lax.broadcasted_iota(
        jnp.int32, (_SP, _SEL), 0).astype(jnp.float32)
    perm_row = jnp.sum(onehot[:, :_SEL] * iota_i,
                       axis=0, keepdims=True)
    gidx_ref[...] = (perm_row.astype(jnp.int32)
                     + n * _S).reshape(1, 1, _SEL)

    sorted_logits = lax.dot_general(
        onehot[:_S, :_S], logits, (((0,), (0,)), ((), ())),
        preferred_element_type=jnp.float32)
    preds1_ref[...] = sorted_logits[:_SEL].reshape(1, _SEL, _K)
    preds0_ref[...] = sorted_logits[_SEL:].reshape(1, _S - _SEL, _K)


def _compute(xt, w, b2d, keys):
    n = xt.shape[0]
    return pl.pallas_call(
        _compute_body,
        grid=(n,),
        in_specs=[
            pl.BlockSpec((1, _S, _C), lambda i: (i, 0, 0)),
            pl.BlockSpec((_K, _C), lambda i: (0, 0)),
            pl.BlockSpec((1, _K), lambda i: (0, 0)),
            pl.BlockSpec((1, _S, 1), lambda i: (i, 0, 0)),
        ],
        out_specs=[
            pl.BlockSpec((1, _SEL, _K), lambda i: (i, 0, 0)),
            pl.BlockSpec((1, _S - _SEL, _K), lambda i: (i, 0, 0)),
            pl.BlockSpec((1, 1, _SEL), lambda i: (i, 0, 0)),
        ],
        out_shape=[
            jax.ShapeDtypeStruct((n, _SEL, _K), jnp.float32),
            jax.ShapeDtypeStruct((n, _S - _SEL, _K), jnp.float32),
            jax.ShapeDtypeStruct((n, 1, _SEL), jnp.int32),
        ],
    )(xt, w, b2d, keys)


_NW = 32
_BSEL = (16 * _SEL) // _NW


def _sc_gather(xt_flat, sel_idx):
    mesh = plsc.VectorSubcoreMesh(core_axis_name="c", subcore_axis_name="s")
    nsel = sel_idx.shape[0]

    @functools.partial(
        pl.kernel,
        mesh=mesh,
        out_type=jax.ShapeDtypeStruct((nsel, _C), jnp.float32),
        scratch_types=[
            pltpu.VMEM((_BSEL,), jnp.int32),
            pltpu.VMEM((_BSEL, _C), jnp.float32),
            pltpu.SemaphoreType.DMA,
        ],
    )
    def run(xt_hbm, sidx_hbm, sel_out, sidx_v, srows_v, sem_s):
        wid = lax.axis_index("s") * 2 + lax.axis_index("c")
        sbase = wid * _BSEL
        pltpu.sync_copy(sidx_hbm.at[pl.ds(sbase, _BSEL)], sidx_v)
        pltpu.async_copy(xt_hbm.at[sidx_v], srows_v, sem_s).wait()
        pltpu.sync_copy(srows_v, sel_out.at[pl.ds(sbase, _BSEL)])

    return run(xt_flat, sel_idx)


def kernel(x, W, b):
    n, c, h, wd = x.shape
    s = h * wd
    xt = jnp.transpose(x.reshape(n, c, s), (0, 2, 1))
    key_logits = xt @ W.T + b
    keys = jnp.max(jax.nn.softmax(key_logits, axis=-1), axis=-1)
    preds1, preds0, gidx = _compute(xt, W, b.reshape(1, _K),
                                    keys.reshape(n, s, 1))
    sel_idx = gidx.reshape(-1)
    sel_rows = _sc_gather(xt.reshape(n * s, c), sel_idx)
    selections = sel_rows.reshape(n, _SEL, c)
    return (selections, preds1, preds0)
